# baseline (speedup 1.0000x reference)
import functools

import jax
import jax.numpy as jnp
from jax import lax
from jax.experimental import pallas as pl
from jax.experimental.pallas import tpu as pltpu
from jax.experimental.pallas import tpu_sc as plsc

_N = 10000
_DF = 128
_R = 6800
_K_TILE = 400
_SC_ROWS = _N - _R
_SLAB = 8
_NSLAB = _SC_ROWS // _SLAB
_NTILES = 79
_NW = 32
_ACC_ROWS = 384


def _tc_partial(x_ref, adj_ref, w_ref, out_ref, *, nk):
    k = pl.program_id(0)
    h = jnp.dot(x_ref[...], w_ref[...],
                preferred_element_type=jnp.float32).astype(jnp.bfloat16)
    contrib = jax.lax.dot_general(
        adj_ref[...].astype(jnp.bfloat16), h,
        (((0,), (0,)), ((), ())),
        preferred_element_type=jnp.float32)

    @pl.when(k == 0)
    def _():
        out_ref[...] = contrib

    @pl.when(k > 0)
    def _():
        out_ref[...] += contrib


def _h_kernel(x_ref, w_ref, h_ref):
    h_ref[...] = jnp.dot(x_ref[...], w_ref[...],
                         preferred_element_type=jnp.float32)


def _combine(a_ref, c_ref, b_ref, out_ref):
    out_ref[...] = jnp.maximum(a_ref[...] + c_ref[...] + b_ref[...], 0.0)


def _iota16():
    return lax.iota(jnp.int32, 16)


def _splat(v):
    return jnp.full((16,), v, jnp.int32)


def _sc_body(adj_hbm, h_hbm, out_hbm,
             acc, hitbuf, counts, a0, a1, h0, h1, sa0, sa1, sh0, sh1):
    c = lax.axis_index("c")
    s = lax.axis_index("s")
    wid = s * 2 + c

    three = wid < 15
    ntiles = jnp.where(three, 3, 2)
    tbase = jnp.where(three, 3 * wid, 45 + 2 * (wid - 15))
    dbase = tbase * 128
    ncols = jnp.minimum(ntiles * 128, _N - dbase)
    nu = (ncols + 15) // 16
    nu8 = (nu + 7) // 8

    abufs = (a0, a1)
    hbufs = (h0, h1)
    asems = (sa0, sa1)
    hsems = (sh0, sh1)

    def adj_src2(j):
        return adj_hbm.at[pl.ds(_R + j * _SLAB, _SLAB), pl.ds(dbase, 256)]

    def adj_src3(j):
        return adj_hbm.at[pl.ds(_R + j * _SLAB, _SLAB),
                          pl.ds(dbase + 256, 128)]

    def h_src(j):
        return h_hbm.at[pl.ds(j * _SLAB, _SLAB), :]

    def fire(j, p):
        pltpu.async_copy(adj_src2(j), abufs[p].at[:, pl.ds(0, 256)],
                         asems[p])
        pltpu.async_copy(h_src(j), hbufs[p], hsems[p])

        @pl.when(three)
        def _():
            pltpu.async_copy(adj_src3(j), abufs[p].at[:, pl.ds(256, 128)],
                             asems[p])

    def drain(p):
        pltpu.make_async_copy(adj_src2(0), abufs[p].at[:, pl.ds(0, 256)],
                              asems[p]).wait()
        pltpu.make_async_copy(h_src(0), hbufs[p], hsems[p]).wait()

        @pl.when(three)
        def _():
            pltpu.make_async_copy(adj_src3(0),
                                  abufs[p].at[:, pl.ds(256, 128)],
                                  asems[p]).wait()

    zf = jnp.zeros((16,), jnp.float32)
    it16 = _iota16()

    def zrow(i, carry):
        for q in range(8):
            plsc.store_scatter(acc, [_splat(i), q * 16 + it16], zf)
        return carry

    lax.fori_loop(0, _ACC_ROWS, zrow, 0)

    nunits = _ACC_ROWS // 16
    valid_m = [jnp.full((16,), u < nu) for u in range(nunits)]
    nchunks = _SLAB * nunits
    ngroups = nchunks // 16
    nu_v = jnp.full((16,), nu, jnp.int32)
    lane_sel = [it16 == l for l in range(16)]

    def process(p, slab):
        ab = abufs[p]
        hb = hbufs[p]
        pres = _splat(0)
        for r in range(_SLAB):
            for u in range(nunits):
                ci = r * nunits + u
                l = ci % 16
                v = ab[r, u * 16:(u + 1) * 16]
                m = jnp.logical_and(v != 0.0, valid_m[u])
                cnt = plsc.all_reduce_population_count(m)
                pres = pres + jnp.where(lane_sel[l], cnt, 0)
                if l == 15:
                    counts[pl.ds((ci // 16) * 16, 16)] = pres
                    pres = _splat(0)

        def chunk_extract(carry):
            mg, pos_v, g16 = carry
            l = plsc.all_reduce_ffs(mg)
            ci = g16 + l
            r = ci // nunits
            u = ci - r * nunits
            v = plsc.load_gather(ab, [r, u * 16 + it16])
            m = jnp.logical_and(v != 0.0, u < nu_v)
            incl = plsc.cumsum(m.astype(jnp.int32))
            idx = pos_v + incl - 1
            val = r * 512 + u * 16 + it16
            plsc.store_scatter(hitbuf, [idx], val, mask=m)
            pos_v = pos_v + plsc.all_reduce_population_count(m)
            mg = jnp.logical_and(mg, jnp.logical_not(lane_sel_dyn(l)))
            return mg, pos_v, g16

        def lane_sel_dyn(l):
            return it16 == l

        def any_left(carry):
            mg = carry[0]
            return jnp.max(plsc.all_reduce_population_count(mg)) > 0

        pos_v = _splat(0)
        for g in range(ngroups):
            cg = counts[pl.ds(g * 16, 16)]
            mg0 = cg > 0
            mg, pos_v, _ = lax.while_loop(
                any_left, chunk_extract, (mg0, pos_v, _splat(g * 16)))
        pos = jnp.max(pos_v)

        def hit_body(i, carry, hb=hb):
            val = plsc.load_gather(hitbuf, [_splat(i)])
            d = jnp.bitwise_and(val, 511)
            r = jnp.right_shift(val, 9)
            for q in range(8):
                fidx = q * 16 + it16
                hq = plsc.load_gather(hb, [r, fidx])
                plsc.addupdate_scatter(acc, [d, fidx], hq)
            return carry

        lax.fori_loop(0, pos, hit_body, 0)

    fire(0, 0)

    def outer(jj, carry):
        j0 = 2 * jj
        fire(j0 + 1, 1)
        drain(0)
        process(0, j0)
        fire(jnp.minimum(j0 + 2, _NSLAB - 1), 0)
        drain(1)
        process(1, j0 + 1)
        return carry

    lax.fori_loop(0, _NSLAB // 2, outer, 0)
    drain(0)

    def wrow(t, carry):
        pltpu.sync_copy(acc.at[pl.ds(t * 16, 16), :],
                        out_hbm.at[pl.ds(dbase + t * 16, 16), :])
        return carry

    lax.fori_loop(0, nu, wrow, 0)


def _sc_partial(adj, h_sc):
    mesh = plsc.VectorSubcoreMesh(core_axis_name="c", subcore_axis_name="s")
    f = pl.kernel(
        _sc_body,
        out_type=jax.ShapeDtypeStruct((_N, _DF), jnp.float32),
        mesh=mesh,
        scratch_types=[
            pltpu.VMEM((_ACC_ROWS, _DF), jnp.float32),
            pltpu.VMEM((3088,), jnp.int32),
            pltpu.VMEM((_SLAB * (_ACC_ROWS // 16),), jnp.int32),
            pltpu.VMEM((_SLAB, _ACC_ROWS), jnp.float32),
            pltpu.VMEM((_SLAB, _ACC_ROWS), jnp.float32),
            pltpu.VMEM((_SLAB, _DF), jnp.float32),
            pltpu.VMEM((_SLAB, _DF), jnp.float32),
            pltpu.SemaphoreType.DMA,
            pltpu.SemaphoreType.DMA,
            pltpu.SemaphoreType.DMA,
            pltpu.SemaphoreType.DMA,
        ],
        compiler_params=pltpu.CompilerParams(use_tc_tiling_on_sc=True,
                                             needs_layout_passes=False),
    )
    return f(adj, h_sc)


def kernel(x, adj, W, b):
    n, d_in = x.shape
    d_out = W.shape[1]
    b2 = b.reshape(1, d_out).astype(jnp.float32)

    h_sc = pl.pallas_call(
        _h_kernel,
        in_specs=[
            pl.BlockSpec((_SC_ROWS, d_in), lambda: (0, 0)),
            pl.BlockSpec((d_in, d_out), lambda: (0, 0)),
        ],
        out_specs=pl.BlockSpec((_SC_ROWS, d_out), lambda: (0, 0)),
        out_shape=jax.ShapeDtypeStruct((_SC_ROWS, d_out), jnp.float32),
    )(x[_R:], W)

    sc_part = _sc_partial(adj, h_sc)

    nk = _R // _K_TILE
    tc_part = pl.pallas_call(
        functools.partial(_tc_partial, nk=nk),
        grid=(nk,),
        in_specs=[
            pl.BlockSpec((_K_TILE, d_in), lambda k: (k, 0)),
            pl.BlockSpec((_K_TILE, n), lambda k: (k, 0)),
            pl.BlockSpec((d_in, d_out), lambda k: (0, 0)),
        ],
        out_specs=pl.BlockSpec((n, d_out), lambda k: (0, 0)),
        out_shape=jax.ShapeDtypeStruct((n, d_out), jnp.float32),
    )(x, adj, W)

    cb = 400
    out = pl.pallas_call(
        _combine,
        grid=(n // cb,),
        in_specs=[
            pl.BlockSpec((cb, d_out), lambda k: (k, 0)),
            pl.BlockSpec((cb, d_out), lambda k: (k, 0)),
            pl.BlockSpec((1, d_out), lambda k: (0, 0)),
        ],
        out_specs=pl.BlockSpec((cb, d_out), lambda k: (k, 0)),
        out_shape=jax.ShapeDtypeStruct((n, d_out), jnp.float32),
    )(tc_part, sc_part, b2)
    return (out, adj)


# speedup vs baseline: 6.2627x; 6.2627x over previous
import functools

import jax
import jax.numpy as jnp
from jax.experimental import pallas as pl


def _gcn_kernel(x_ref, adj_ref, w_ref, b_ref, out_ref, *, nk):
    k = pl.program_id(0)
    h = jnp.dot(x_ref[...], w_ref[...], preferred_element_type=jnp.float32)
    contrib = jax.lax.dot_general(
        adj_ref[...].astype(jnp.bfloat16), h.astype(jnp.bfloat16),
        (((0,), (0,)), ((), ())),
        preferred_element_type=jnp.float32)

    @pl.when(k == 0)
    def _():
        out_ref[...] = contrib

    @pl.when(k > 0)
    def _():
        out_ref[...] += contrib

    @pl.when(k == nk - 1)
    def _():
        out_ref[...] = jnp.maximum(out_ref[...] + b_ref[...], 0.0)


def kernel(x, adj, W, b):
    n, d_in = x.shape
    d_out = W.shape[1]

    k_tile = 400
    if n % k_tile:
        k_tile = n
    nk = n // k_tile

    b2 = b.reshape(1, d_out).astype(jnp.float32)

    out = pl.pallas_call(
        functools.partial(_gcn_kernel, nk=nk),
        grid=(nk,),
        in_specs=[
            pl.BlockSpec((k_tile, d_in), lambda k: (k, 0)),
            pl.BlockSpec((k_tile, n), lambda k: (k, 0)),
            pl.BlockSpec((d_in, d_out), lambda k: (0, 0)),
            pl.BlockSpec((1, d_out), lambda k: (0, 0)),
        ],
        out_specs=pl.BlockSpec((n, d_out), lambda k: (0, 0)),
        out_shape=jax.ShapeDtypeStruct((n, d_out), jnp.float32),
    )(x, adj, W, b2)
    return (out, adj)
